# Initial kernel scaffold; baseline (speedup 1.0000x reference)
import jax
import jax.numpy as jnp
from jax.experimental import pallas as pl


def kernel(x, eic, eid, eit, batch, Wl1, Wr1, b1, Wl2, Wr2, b2, Wl3, Wr3, b3, Wl4, Wr4, b4, Wl5, Wr5, b5, Wm1, bm1, Wm2, bm2, Wo, bo):
    raise NotImplementedError("write your pallas kernel here")


# speedup vs baseline: 6.3909x; 6.3909x over previous
import functools

import jax
import jax.numpy as jnp
from jax import lax
from jax.experimental import pallas as pl
from jax.experimental.pallas import tpu as pltpu
from jax.experimental.pallas import tpu_sc as plsc

N = 10000
D = 128
E = 320000
G = 64
NW = 32
EPW = E // NW
CH = 80
NCH = EPW // CH
RB = 1000
NB = N // RB
NP = 10240

_MESH = plsc.VectorSubcoreMesh(core_axis_name="c", subcore_axis_name="s")


@functools.partial(
    pl.kernel,
    out_type=jax.ShapeDtypeStruct((2, N, D), jnp.float32),
    mesh=_MESH,
    scratch_types=[
        pltpu.VMEM((NCH, CH), jnp.int32),
        pltpu.VMEM((NCH, CH), jnp.int32),
        pltpu.VMEM((CH, D), jnp.float32),
        pltpu.VMEM((40, D), jnp.float32),
        pltpu.VMEM_SHARED((N, D), jnp.float32),
        pltpu.SemaphoreType.DMA,
    ],
)
def _sc_agg(u_hbm, src_hbm, dst_hbm, out_hbm, idx_s, idx_d, rows, zbuf, acc, sem):
    c = lax.axis_index("c")
    s = lax.axis_index("s")
    wid = s * 2 + c
    z16 = jnp.zeros((16,), jnp.float32)

    @pl.loop(0, 40)
    def _(i):
        for j in range(8):
            zbuf[i, pl.ds(j * 16, 16)] = z16

    @pl.loop(0, 25)
    def _(k):
        @pl.when((k % 16) == s)
        def _():
            for t in range(10):
                pltpu.sync_copy(zbuf, acc.at[pl.ds(k * 400 + t * 40, 40)])

    plsc.subcore_barrier()

    pltpu.sync_copy(src_hbm.at[wid], idx_s)
    pltpu.sync_copy(dst_hbm.at[wid], idx_d)

    @pl.loop(0, NCH)
    def _(j):
        pltpu.async_copy(u_hbm.at[idx_s.at[j]], rows, sem).wait()
        pltpu.sync_copy(rows, acc.at[idx_d.at[j]], add=True)

    plsc.subcore_barrier()

    @pl.loop(0, 25)
    def _(k):
        @pl.when((k % 16) == s)
        def _():
            pltpu.sync_copy(acc.at[pl.ds(k * 400, 400)],
                            out_hbm.at[c, pl.ds(k * 400, 400)])


@functools.partial(
    pl.kernel,
    out_type=(jax.ShapeDtypeStruct((2 * NP,), jnp.float32),
              jax.ShapeDtypeStruct((2 * NP,), jnp.float32)),
    mesh=_MESH,
    scratch_types=[
        pltpu.VMEM((NCH, CH), jnp.int32),
        pltpu.VMEM((CH,), jnp.float32),
        pltpu.VMEM((512,), jnp.float32),
        pltpu.VMEM_SHARED((NP,), jnp.float32),
        pltpu.VMEM_SHARED((NP,), jnp.float32),
    ],
)
def _sc_deg(dstd_hbm, dstc_hbm, outd_hbm, outc_hbm, idx_d, ones_v, zv, accd, accc):
    c = lax.axis_index("c")
    s = lax.axis_index("s")
    wid = s * 2 + c
    z16 = jnp.zeros((16,), jnp.float32)
    o16 = jnp.ones((16,), jnp.float32)

    @pl.loop(0, 32)
    def _(i):
        zv[pl.ds(i * 16, 16)] = z16

    @pl.loop(0, 5)
    def _(i):
        ones_v[pl.ds(i * 16, 16)] = o16

    @pl.loop(0, 20)
    def _(k):
        @pl.when((k % 16) == s)
        def _():
            pltpu.sync_copy(zv, accd.at[pl.ds(k * 512, 512)])
            pltpu.sync_copy(zv, accc.at[pl.ds(k * 512, 512)])

    plsc.subcore_barrier()

    pltpu.sync_copy(dstd_hbm.at[wid], idx_d)

    @pl.loop(0, NCH)
    def _(j):
        pltpu.sync_copy(ones_v, accd.at[idx_d.at[j]], add=True)

    pltpu.sync_copy(dstc_hbm.at[wid], idx_d)

    @pl.loop(0, NCH)
    def _(j):
        pltpu.sync_copy(ones_v, accc.at[idx_d.at[j]], add=True)

    plsc.subcore_barrier()

    @pl.loop(0, 20)
    def _(k):
        @pl.when((k % 16) == s)
        def _():
            pltpu.sync_copy(accd.at[pl.ds(k * 512, 512)],
                            outd_hbm.at[pl.ds(c * NP + k * 512, 512)])
            pltpu.sync_copy(accc.at[pl.ds(k * 512, 512)],
                            outc_hbm.at[pl.ds(c * NP + k * 512, 512)])


def _tc_layer_mean_body(p_ref, dg_ref, hp_ref, wl_ref, wr_ref, b_ref, h_ref):
    a = (p_ref[0] + p_ref[1]) / jnp.maximum(dg_ref[0] + dg_ref[1], 1.0)
    h_ref[...] = (jnp.dot(a, wl_ref[...], preferred_element_type=jnp.float32)
                  + jnp.dot(hp_ref[...], wr_ref[...], preferred_element_type=jnp.float32)
                  + b_ref[...])


def _tc_layer_mean(p, dg, hp, wl, wr, b):
    return pl.pallas_call(
        _tc_layer_mean_body,
        grid=(NB,),
        in_specs=[pl.BlockSpec((2, RB, D), lambda i: (0, i, 0)),
                  pl.BlockSpec((2, RB, 1), lambda i: (0, i, 0)),
                  pl.BlockSpec((RB, D), lambda i: (i, 0)),
                  pl.BlockSpec((D, D), lambda i: (0, 0)),
                  pl.BlockSpec((D, D), lambda i: (0, 0)),
                  pl.BlockSpec((1, D), lambda i: (0, 0))],
        out_specs=pl.BlockSpec((RB, D), lambda i: (i, 0)),
        out_shape=jax.ShapeDtypeStruct((N, D), jnp.float32),
    )(p, dg, hp, wl, wr, b)


def _tc_layer_sum_body(p_ref, hp_ref, wl_ref, wr_ref, b_ref, h_ref):
    a = p_ref[0] + p_ref[1]
    h_ref[...] = (jnp.dot(a, wl_ref[...], preferred_element_type=jnp.float32)
                  + jnp.dot(hp_ref[...], wr_ref[...], preferred_element_type=jnp.float32)
                  + b_ref[...])


def _tc_layer_sum(p, hp, wl, wr, b):
    return pl.pallas_call(
        _tc_layer_sum_body,
        grid=(NB,),
        in_specs=[pl.BlockSpec((2, RB, D), lambda i: (0, i, 0)),
                  pl.BlockSpec((RB, D), lambda i: (i, 0)),
                  pl.BlockSpec((D, D), lambda i: (0, 0)),
                  pl.BlockSpec((D, D), lambda i: (0, 0)),
                  pl.BlockSpec((1, D), lambda i: (0, 0))],
        out_specs=pl.BlockSpec((RB, D), lambda i: (i, 0)),
        out_shape=jax.ShapeDtypeStruct((N, D), jnp.float32),
    )(p, hp, wl, wr, b)


def _tc_pool_body(h_ref, bt_ref, wm1_ref, bm1_ref, wm2_ref,
                  bm2_ref, wo_ref, bo_ref, o_ref, acc):
    i = pl.program_id(0)

    @pl.when(i == 0)
    def _():
        acc[...] = jnp.zeros_like(acc)

    bt = bt_ref[0]
    oh = (bt == lax.broadcasted_iota(jnp.int32, (G, RB), 0)).astype(jnp.float32)
    acc[...] += jnp.dot(oh, h_ref[...], preferred_element_type=jnp.float32,
                        precision=lax.Precision.HIGHEST)

    @pl.when(i == NB - 1)
    def _():
        g = acc[...]
        g = jnp.maximum(jnp.dot(g, wm1_ref[...], preferred_element_type=jnp.float32) + bm1_ref[...], 0.0)
        g = jnp.maximum(jnp.dot(g, wm2_ref[...], preferred_element_type=jnp.float32) + bm2_ref[...], 0.0)
        o_ref[...] = jnp.dot(g, wo_ref[...], preferred_element_type=jnp.float32) + bo_ref[...]


def _tc_pool(h, bt3, wm1, bm1, wm2, bm2, wo, bo):
    return pl.pallas_call(
        _tc_pool_body,
        grid=(NB,),
        in_specs=[pl.BlockSpec((RB, D), lambda i: (i, 0)),
                  pl.BlockSpec((1, 1, RB), lambda i: (i, 0, 0)),
                  pl.BlockSpec((D, D), lambda i: (0, 0)),
                  pl.BlockSpec((1, D), lambda i: (0, 0)),
                  pl.BlockSpec((D, D), lambda i: (0, 0)),
                  pl.BlockSpec((1, D), lambda i: (0, 0)),
                  pl.BlockSpec((D, 1), lambda i: (0, 0)),
                  pl.BlockSpec((1, 1), lambda i: (0, 0))],
        out_specs=pl.BlockSpec((G, 1), lambda i: (0, 0)),
        out_shape=jax.ShapeDtypeStruct((G, 1), jnp.float32),
        scratch_shapes=[pltpu.VMEM((G, D), jnp.float32)],
    )(h, bt3, wm1, bm1, wm2, bm2, wo, bo)


def kernel(x, eic, eid, eit, batch, Wl1, Wr1, b1, Wl2, Wr2, b2, Wl3, Wr3, b3,
           Wl4, Wr4, b4, Wl5, Wr5, b5, Wm1, bm1, Wm2, bm2, Wo, bo):
    srcd = eid[0].reshape(NW, NCH, CH)
    dstd = eid[1].reshape(NW, NCH, CH)
    srcc = eic[0].reshape(NW, NCH, CH)
    dstc = eic[1].reshape(NW, NCH, CH)
    srct = eit[0].reshape(NW, NCH, CH)
    dstt = eit[1].reshape(NW, NCH, CH)
    bt3 = batch.reshape(NB, 1, RB)
    b1r, b2r, b3r = b1.reshape(1, D), b2.reshape(1, D), b3.reshape(1, D)
    b4r, b5r = b4.reshape(1, D), b5.reshape(1, D)

    degd, degc = _sc_deg(dstd, dstc)
    x, degd, degc = lax.optimization_barrier((x, degd, degc))
    degd3 = degd.reshape(2, NP, 1)[:, :N]
    degc3 = degc.reshape(2, NP, 1)[:, :N]

    p = _sc_agg(x, srcd, dstd)
    h = _tc_layer_mean(p, degd3, x, Wl1, Wr1, b1r)
    p = _sc_agg(h, srcc, dstc)
    h = _tc_layer_mean(p, degc3, h, Wl2, Wr2, b2r)
    p = _sc_agg(h, srcc, dstc)
    h = _tc_layer_mean(p, degc3, h, Wl2, Wr2, b2r)
    p = _sc_agg(h, srct, dstt)
    h = _tc_layer_sum(p, h, Wl3, Wr3, b3r)
    p = _sc_agg(h, srcd, dstd)
    h = _tc_layer_mean(p, degd3, h, Wl4, Wr4, b4r)
    p = _sc_agg(h, srcc, dstc)
    h = _tc_layer_mean(p, degc3, h, Wl5, Wr5, b5r)
    p = _sc_agg(h, srcc, dstc)
    h = _tc_layer_mean(p, degc3, h, Wl5, Wr5, b5r)
    return _tc_pool(h, bt3, Wm1, bm1.reshape(1, D), Wm2,
                    bm2.reshape(1, D), Wo, bo.reshape(1, 1))
